# baseline (speedup 1.0000x reference)
import functools
import jax
import jax.numpy as jnp
from jax import lax
from jax.experimental import pallas as pl
from jax.experimental.pallas import tpu as pltpu
from jax.experimental.pallas import tpu_sc as plsc

N_CLASS = 1000000
EMBED_DIM = 32
BATCH = 16384
FIELDS = 26

NC = 2
NS = 16
NW = NC * NS

RBLK = 256
GBLK = RBLK // 4
N_BLK = N_CLASS // RBLK
A_ITER = (N_BLK + NW - 1) // NW
R_TAIL = N_CLASS - N_BLK * RBLK
G_TAIL = R_TAIL // 4

S_PER_W = BATCH // NW
S_PER_CHUNK = 8
CHUNK = S_PER_CHUNK * FIELDS
N_CHUNK = S_PER_W // S_PER_CHUNK
STREAM = CHUNK // 2

_PARAMS = pltpu.CompilerParams(
    needs_layout_passes=False, use_tc_tiling_on_sc=True
)


def _compact(xin, tout, nrows):
    for r in range(nrows):
        g, o = r // 4, 32 * (r % 4)
        tout[g, pl.ds(o, 16)] = xin[r, pl.ds(0, 16)]
        tout[g, pl.ds(o + 16, 16)] = xin[r, pl.ds(16, 16)]


def _repack_body(table_hbm, t128_hbm, xin0, tout0, xin1, tout1, tailx, tailt, sem):
    wid = lax.axis_index("s") * NC + lax.axis_index("c")

    def do_block(b, xin, tout):
        @pl.when(b < N_BLK)
        def _():
            pltpu.sync_copy(table_hbm.at[pl.ds(b * RBLK, RBLK)], xin)
            _compact(xin, tout, RBLK)
            pltpu.sync_copy(tout, t128_hbm.at[pl.ds(b * GBLK, GBLK)])

    def pair_body(i, carry):
        do_block(wid + NW * (2 * i), xin0, tout0)
        do_block(wid + NW * (2 * i + 1), xin1, tout1)
        return carry

    lax.fori_loop(0, (A_ITER + 1) // 2, pair_body, 0, unroll=False)

    @pl.when(wid == 0)
    def _():
        pltpu.sync_copy(table_hbm.at[pl.ds(N_BLK * RBLK, R_TAIL)], tailx)
        _compact(tailx, tailt, R_TAIL)
        pltpu.sync_copy(tailt, t128_hbm.at[pl.ds(N_BLK * GBLK, G_TAIL)])


def _fire(x_hbm, t128_hbm, s0, xv, gv, rows_v, sem):
    iota = lax.iota(jnp.int32, 16)
    pltpu.sync_copy(x_hbm.at[pl.ds(s0, S_PER_CHUNK)], xv)
    for si in range(S_PER_CHUNK):
        v1 = xv[si, pl.ds(0, 16)] >> 2
        v2 = xv[si, pl.ds(10, 16)] >> 2
        plsc.store_scatter(gv, [iota + (FIELDS * si)], v1)
        plsc.store_scatter(gv, [iota + (FIELDS * si + 10)], v2)
    copies = []
    for j in range(2):
        copies.append(
            pltpu.async_copy(
                t128_hbm.at[gv.at[pl.ds(j * STREAM, STREAM)]],
                rows_v.at[pl.ds(j * STREAM, STREAM)],
                sem,
            )
        )
    return copies


def _drain_extract(out_hbm, s0, copies, xv, rows_v, out_v, osem):
    for c in copies:
        c.wait()
    iota = lax.iota(jnp.int32, 16)
    for si in range(S_PER_CHUNK):
        x1 = xv[si, pl.ds(0, 16)]
        o1 = (x1 & 3) << 5
        x2 = xv[si, pl.ds(10, 16)]
        o2 = (x2 & 3) << 5
        for f in range(FIELDS):
            if f < 16:
                osp = jnp.take(o1, jnp.full((16,), f, jnp.int32))
            else:
                osp = jnp.take(o2, jnp.full((16,), f - 10, jnp.int32))
            j = FIELDS * si + f
            jv = jnp.full((16,), j, jnp.int32)
            out_v[j, pl.ds(0, 16)] = plsc.load_gather(
                rows_v, [jv, osp + iota])
            out_v[j, pl.ds(16, 16)] = plsc.load_gather(
                rows_v, [jv, osp + iota + 16])
    ocopies = []
    for si in range(S_PER_CHUNK):
        ocopies.append(
            pltpu.async_copy(
                out_v.at[pl.ds(FIELDS * si, FIELDS)],
                out_hbm.at[s0 + si],
                osem,
            )
        )
    for c in ocopies:
        c.wait()


def _emb_body(
    x_hbm, t128_hbm, out_hbm,
    xv0, gv0, rows0, outv0, xv1, gv1, rows1, outv1, sem0, sem1, osem,
):
    wid = lax.axis_index("s") * NC + lax.axis_index("c")
    sbase = wid * S_PER_W

    def pair_body(ci, carry):
        s0 = sbase + (2 * ci) * S_PER_CHUNK
        s1 = sbase + (2 * ci + 1) * S_PER_CHUNK
        cp0 = _fire(x_hbm, t128_hbm, s0, xv0, gv0, rows0, sem0)
        cp1 = _fire(x_hbm, t128_hbm, s1, xv1, gv1, rows1, sem1)
        _drain_extract(out_hbm, s0, cp0, xv0, rows0, outv0, osem)
        _drain_extract(out_hbm, s1, cp1, xv1, rows1, outv1, osem)
        return carry

    lax.fori_loop(0, N_CHUNK // 2, pair_body, 0, unroll=False)


@jax.jit
def kernel(X, table):
    xi = X.astype(jnp.int32)
    mesh = plsc.VectorSubcoreMesh(core_axis_name="c", subcore_axis_name="s")
    repack = functools.partial(
        pl.kernel,
        mesh=mesh,
        out_type=jax.ShapeDtypeStruct((N_CLASS // 4, 128), jnp.float32),
        compiler_params=_PARAMS,
        scratch_types=[
            pltpu.VMEM((RBLK, EMBED_DIM), jnp.float32),
            pltpu.VMEM((GBLK, 128), jnp.float32),
            pltpu.VMEM((RBLK, EMBED_DIM), jnp.float32),
            pltpu.VMEM((GBLK, 128), jnp.float32),
            pltpu.VMEM((R_TAIL, EMBED_DIM), jnp.float32),
            pltpu.VMEM((G_TAIL, 128), jnp.float32),
            pltpu.SemaphoreType.DMA,
        ],
    )(_repack_body)
    t128 = repack(table)
    lookup = functools.partial(
        pl.kernel,
        mesh=mesh,
        out_type=jax.ShapeDtypeStruct((BATCH, FIELDS, EMBED_DIM), jnp.float32),
        compiler_params=_PARAMS,
        scratch_types=[
            pltpu.VMEM((S_PER_CHUNK, FIELDS), jnp.int32),
            pltpu.VMEM((CHUNK,), jnp.int32),
            pltpu.VMEM((CHUNK, 128), jnp.float32),
            pltpu.VMEM((CHUNK, EMBED_DIM), jnp.float32),
            pltpu.VMEM((S_PER_CHUNK, FIELDS), jnp.int32),
            pltpu.VMEM((CHUNK,), jnp.int32),
            pltpu.VMEM((CHUNK, 128), jnp.float32),
            pltpu.VMEM((CHUNK, EMBED_DIM), jnp.float32),
            pltpu.SemaphoreType.DMA,
            pltpu.SemaphoreType.DMA,
            pltpu.SemaphoreType.DMA,
        ],
    )(_emb_body)
    return lookup(xi, t128)


# speedup vs baseline: 1.2308x; 1.2308x over previous
import functools
import jax
import jax.numpy as jnp
from jax import lax
from jax.experimental import pallas as pl
from jax.experimental.pallas import tpu as pltpu
from jax.experimental.pallas import tpu_sc as plsc

N_CLASS = 1000000
EMBED_DIM = 32
BATCH = 16384
FIELDS = 26

NC = 2
NS = 16
NW = NC * NS

RBLK = 256
GBLK = RBLK // 4
N_BLK = N_CLASS // RBLK
A_ITER = (N_BLK + NW - 1) // NW
R_TAIL = N_CLASS - N_BLK * RBLK
G_TAIL = R_TAIL // 4

S_PER_W = BATCH // NW
S_PER_CHUNK = 8
CHUNK = S_PER_CHUNK * FIELDS
N_CHUNK = S_PER_W // S_PER_CHUNK
STREAM = CHUNK // 2

_PARAMS = pltpu.CompilerParams(
    needs_layout_passes=False, use_tc_tiling_on_sc=True
)


def _compact(xin, tout, ngroups):
    @plsc.parallel_loop(0, ngroups)
    def _(g):
        for k in range(4):
            tout[g, pl.ds(32 * k, 16)] = xin[4 * g + k, pl.ds(0, 16)]
            tout[g, pl.ds(32 * k + 16, 16)] = xin[4 * g + k, pl.ds(16, 16)]


def _repack_body(table_hbm, t128_hbm, xin0, tout0, xin1, tout1, tailx, tailt, sem):
    wid = lax.axis_index("s") * NC + lax.axis_index("c")

    def fire_in(b, xin, sem):
        @pl.when(b < N_BLK)
        def _():
            pltpu.async_copy(table_hbm.at[pl.ds(b * RBLK, RBLK)], xin, sem)

    def do_block(b, xin, tout, sem):
        @pl.when(b < N_BLK)
        def _():
            pltpu.make_async_copy(
                table_hbm.at[pl.ds(b * RBLK, RBLK)], xin, sem
            ).wait()
            _compact(xin, tout, GBLK)
            pltpu.sync_copy(tout, t128_hbm.at[pl.ds(b * GBLK, GBLK)])

    def pair_body(i, carry):
        b0 = wid + NW * (2 * i)
        b1 = wid + NW * (2 * i + 1)
        fire_in(b0, xin0, sem)
        fire_in(b1, xin1, sem)
        do_block(b0, xin0, tout0, sem)
        do_block(b1, xin1, tout1, sem)
        return carry

    lax.fori_loop(0, (A_ITER + 1) // 2, pair_body, 0, unroll=False)

    @pl.when(wid == 0)
    def _():
        pltpu.sync_copy(table_hbm.at[pl.ds(N_BLK * RBLK, R_TAIL)], tailx)
        _compact(tailx, tailt, G_TAIL)
        pltpu.sync_copy(tailt, t128_hbm.at[pl.ds(N_BLK * GBLK, G_TAIL)])


def _fire(x_hbm, t128_hbm, s0, xv, gv, rows_v, sem):
    iota = lax.iota(jnp.int32, 16)
    pltpu.sync_copy(x_hbm.at[pl.ds(s0, S_PER_CHUNK)], xv)
    for si in range(S_PER_CHUNK):
        v1 = xv[si, pl.ds(0, 16)] >> 2
        v2 = xv[si, pl.ds(10, 16)] >> 2
        plsc.store_scatter(gv, [iota + (FIELDS * si)], v1)
        plsc.store_scatter(gv, [iota + (FIELDS * si + 10)], v2)
    copies = []
    for j in range(2):
        copies.append(
            pltpu.async_copy(
                t128_hbm.at[gv.at[pl.ds(j * STREAM, STREAM)]],
                rows_v.at[pl.ds(j * STREAM, STREAM)],
                sem,
            )
        )
    return copies


def _drain_extract(out_hbm, s0, copies, xv, rows_v, out_v, osem):
    for c in copies:
        c.wait()
    iota = lax.iota(jnp.int32, 16)

    @plsc.parallel_loop(0, S_PER_CHUNK)
    def _(si):
        x1 = xv[si, pl.ds(0, 16)]
        o1 = (x1 & 3) << 5
        x2 = xv[si, pl.ds(10, 16)]
        o2 = (x2 & 3) << 5
        for f in range(FIELDS):
            if f < 16:
                osp = jnp.take(o1, jnp.full((16,), f, jnp.int32))
            else:
                osp = jnp.take(o2, jnp.full((16,), f - 10, jnp.int32))
            j = FIELDS * si + f
            jv = jnp.full((16,), j, jnp.int32)
            out_v[j, pl.ds(0, 16)] = plsc.load_gather(
                rows_v, [jv, osp + iota])
            out_v[j, pl.ds(16, 16)] = plsc.load_gather(
                rows_v, [jv, osp + iota + 16])
    ocopies = []
    for si in range(S_PER_CHUNK):
        ocopies.append(
            pltpu.async_copy(
                out_v.at[pl.ds(FIELDS * si, FIELDS)],
                out_hbm.at[s0 + si],
                osem,
            )
        )
    for c in ocopies:
        c.wait()


def _emb_body(
    x_hbm, t128_hbm, out_hbm,
    xv0, gv0, rows0, outv0, xv1, gv1, rows1, outv1, sem0, sem1, osem,
):
    wid = lax.axis_index("s") * NC + lax.axis_index("c")
    sbase = wid * S_PER_W

    def pair_body(ci, carry):
        s0 = sbase + (2 * ci) * S_PER_CHUNK
        s1 = sbase + (2 * ci + 1) * S_PER_CHUNK
        cp0 = _fire(x_hbm, t128_hbm, s0, xv0, gv0, rows0, sem0)
        cp1 = _fire(x_hbm, t128_hbm, s1, xv1, gv1, rows1, sem1)
        _drain_extract(out_hbm, s0, cp0, xv0, rows0, outv0, osem)
        _drain_extract(out_hbm, s1, cp1, xv1, rows1, outv1, osem)
        return carry

    lax.fori_loop(0, N_CHUNK // 2, pair_body, 0, unroll=False)


@jax.jit
def kernel(X, table):
    xi = X.astype(jnp.int32)
    mesh = plsc.VectorSubcoreMesh(core_axis_name="c", subcore_axis_name="s")
    repack = functools.partial(
        pl.kernel,
        mesh=mesh,
        out_type=jax.ShapeDtypeStruct((N_CLASS // 4, 128), jnp.float32),
        compiler_params=_PARAMS,
        scratch_types=[
            pltpu.VMEM((RBLK, EMBED_DIM), jnp.float32),
            pltpu.VMEM((GBLK, 128), jnp.float32),
            pltpu.VMEM((RBLK, EMBED_DIM), jnp.float32),
            pltpu.VMEM((GBLK, 128), jnp.float32),
            pltpu.VMEM((R_TAIL, EMBED_DIM), jnp.float32),
            pltpu.VMEM((G_TAIL, 128), jnp.float32),
            pltpu.SemaphoreType.DMA,
        ],
    )(_repack_body)
    t128 = repack(table)
    lookup = functools.partial(
        pl.kernel,
        mesh=mesh,
        out_type=jax.ShapeDtypeStruct((BATCH, FIELDS, EMBED_DIM), jnp.float32),
        compiler_params=_PARAMS,
        scratch_types=[
            pltpu.VMEM((S_PER_CHUNK, FIELDS), jnp.int32),
            pltpu.VMEM((CHUNK,), jnp.int32),
            pltpu.VMEM((CHUNK, 128), jnp.float32),
            pltpu.VMEM((CHUNK, EMBED_DIM), jnp.float32),
            pltpu.VMEM((S_PER_CHUNK, FIELDS), jnp.int32),
            pltpu.VMEM((CHUNK,), jnp.int32),
            pltpu.VMEM((CHUNK, 128), jnp.float32),
            pltpu.VMEM((CHUNK, EMBED_DIM), jnp.float32),
            pltpu.SemaphoreType.DMA,
            pltpu.SemaphoreType.DMA,
            pltpu.SemaphoreType.DMA,
        ],
    )(_emb_body)
    return lookup(xi, t128)


# speedup vs baseline: 1.4316x; 1.1631x over previous
import functools
import jax
import jax.numpy as jnp
from jax import lax
from jax.experimental import pallas as pl
from jax.experimental.pallas import tpu as pltpu
from jax.experimental.pallas import tpu_sc as plsc

N_CLASS = 1000000
EMBED_DIM = 32
BATCH = 16384
FIELDS = 26

NC = 2
NS = 16
NW = NC * NS
S_PER_W = BATCH // NW
S_PER_CHUNK = 16
CHUNK = S_PER_CHUNK * FIELDS
N_CHUNK = S_PER_W // S_PER_CHUNK
STREAM = 104
N_STREAM = CHUNK // STREAM


def _fire(x_hbm, table_hbm, s0, xv, gv, rows_v, sem):
    iota = lax.iota(jnp.int32, 16)
    pltpu.sync_copy(x_hbm.at[pl.ds(s0, S_PER_CHUNK)], xv)
    for si in range(S_PER_CHUNK):
        v1 = xv[si, pl.ds(0, 16)]
        v2 = xv[si, pl.ds(10, 16)]
        plsc.store_scatter(gv, [iota + (FIELDS * si)], v1)
        plsc.store_scatter(gv, [iota + (FIELDS * si + 10)], v2)
    copies = []
    for j in range(N_STREAM):
        copies.append(
            pltpu.async_copy(
                table_hbm.at[gv.at[pl.ds(j * STREAM, STREAM)]],
                rows_v.at[pl.ds(j * STREAM, STREAM)],
                sem,
            )
        )
    return copies


def _drain_store(out_hbm, s0, copies, rows_v, osem):
    for c in copies:
        c.wait()
    ocopies = []
    for si in range(S_PER_CHUNK):
        ocopies.append(
            pltpu.async_copy(
                rows_v.at[pl.ds(FIELDS * si, FIELDS)],
                out_hbm.at[s0 + si],
                osem,
            )
        )
    return ocopies


def _emb_body(
    x_hbm, table_hbm, out_hbm,
    xv0, gv0, rows0, xv1, gv1, rows1, sem0, sem1, osem,
):
    wid = lax.axis_index("s") * NC + lax.axis_index("c")
    sbase = wid * S_PER_W

    def pair_body(ci, carry):
        s0 = sbase + (2 * ci) * S_PER_CHUNK
        s1 = sbase + (2 * ci + 1) * S_PER_CHUNK
        cp0 = _fire(x_hbm, table_hbm, s0, xv0, gv0, rows0, sem0)
        cp1 = _fire(x_hbm, table_hbm, s1, xv1, gv1, rows1, sem1)
        ocp0 = _drain_store(out_hbm, s0, cp0, rows0, osem)
        ocp1 = _drain_store(out_hbm, s1, cp1, rows1, osem)
        for c in ocp0 + ocp1:
            c.wait()
        return carry

    lax.fori_loop(0, N_CHUNK // 2, pair_body, 0, unroll=False)


@jax.jit
def kernel(X, table):
    xi = X.astype(jnp.int32)
    mesh = plsc.VectorSubcoreMesh(core_axis_name="c", subcore_axis_name="s")
    f = functools.partial(
        pl.kernel,
        mesh=mesh,
        out_type=jax.ShapeDtypeStruct((BATCH, FIELDS, EMBED_DIM), jnp.float32),
        compiler_params=pltpu.CompilerParams(
            needs_layout_passes=False, use_tc_tiling_on_sc=False
        ),
        scratch_types=[
            pltpu.VMEM((S_PER_CHUNK, FIELDS), jnp.int32),
            pltpu.VMEM((CHUNK,), jnp.int32),
            pltpu.VMEM((CHUNK, EMBED_DIM), jnp.float32),
            pltpu.VMEM((S_PER_CHUNK, FIELDS), jnp.int32),
            pltpu.VMEM((CHUNK,), jnp.int32),
            pltpu.VMEM((CHUNK, EMBED_DIM), jnp.float32),
            pltpu.SemaphoreType.DMA,
            pltpu.SemaphoreType.DMA,
            pltpu.SemaphoreType.DMA,
        ],
    )(_emb_body)
    return f(xi, table)
